# Initial kernel scaffold; baseline (speedup 1.0000x reference)
import jax
import jax.numpy as jnp
from jax.experimental import pallas as pl


def kernel(normu, cls):
    raise NotImplementedError("write your pallas kernel here")


# speedup vs baseline: 11.2148x; 11.2148x over previous
import functools

import jax
import jax.numpy as jnp
from jax import lax
from jax.experimental import pallas as pl
from jax.experimental.pallas import tpu as pltpu
from jax.experimental.pallas import tpu_sc as plsc

N_ROWS = 32
N_COLS = 100000
K = 8
INV_T = 0.5
L = 16
U = 10
CHUNK = U * L
N_STEPS = N_COLS // CHUNK
CPL = 256
CAND = CPL * L
BIG_NEG = -3.0e38
I32_MAX = 2**31 - 1

_mesh = plsc.VectorSubcoreMesh(core_axis_name="c", subcore_axis_name="s")


@functools.partial(
    pl.kernel,
    mesh=_mesh,
    compiler_params=pltpu.CompilerParams(needs_layout_passes=False),
    out_type=jax.ShapeDtypeStruct((N_ROWS, N_COLS), jnp.float32),
    scratch_types=[
        pltpu.VMEM((N_COLS,), jnp.float32),
        pltpu.VMEM((CAND,), jnp.float32),
        pltpu.VMEM((CAND,), jnp.int32),
    ],
)
def _diff_topk_rows(cls_hbm, out_hbm, row, cand_v, cand_i):
    cid = lax.axis_index("c")
    sid = lax.axis_index("s")
    wid = sid * 2 + cid

    pltpu.sync_copy(cls_hbm.at[wid], row)

    lanes = lax.iota(jnp.int32, L)

    def amax_body(i, ms):
        m0, m1 = ms
        base = i * CHUNK
        for u in range(0, U, 2):
            m0 = jnp.maximum(m0, row[pl.ds(base + u * L, L)])
            m1 = jnp.maximum(m1, row[pl.ds(base + (u + 1) * L, L)])
        return (m0, m1)

    neg = jnp.full((L,), BIG_NEG, jnp.float32)
    m0, m1 = lax.fori_loop(0, N_STEPS, amax_body, (neg, neg))
    m_lane = jnp.maximum(m0, m1)
    mg = jnp.max(m_lane)
    mv = m_lane
    for _ in range(K - 1):
        cur = jnp.max(mv)
        mv = jnp.where(mv == cur, neg, mv)
    thr = jnp.max(mv)

    imax_v = jnp.full((L,), I32_MAX, jnp.int32)

    def init_body(i, c):
        cand_v[pl.ds(i * L, L)] = neg
        cand_i[pl.ds(i * L, L)] = imax_v
        return c

    lax.fori_loop(0, CAND // L, init_body, 0)

    def collect_body(i, carry):
        a0, a1, lcnt = carry
        base = i * CHUNK
        for u in range(U):
            off = base + u * L
            v = row[pl.ds(off, L)]
            e = jnp.exp((v - mg) * INV_T)
            if u % 2 == 0:
                a0 = a0 + e
            else:
                a1 = a1 + e
            msk = v >= thr
            pos = jnp.minimum(lcnt, CPL - 1) * L + lanes
            plsc.store_scatter(cand_v, [pos], v, mask=msk)
            plsc.store_scatter(cand_i, [pos], off + lanes, mask=msk)
            lcnt = lcnt + msk.astype(jnp.int32)
        return (a0, a1, lcnt)

    zf = jnp.zeros((L,), jnp.float32)
    zi = jnp.zeros((L,), jnp.int32)
    a0, a1, lcnt = lax.fori_loop(0, N_STEPS, collect_body, (zf, zf, zi))
    s_total = jnp.sum(a0 + a1)
    n_slots = jnp.max(jnp.minimum(lcnt, CPL))

    chosen_v = []
    chosen_i = []
    for j in range(K):

        def sel_body(c, carry, _chosen_i=tuple(chosen_i)):
            bv, bi = carry
            v = cand_v[pl.ds(c * L, L)]
            ii = cand_i[pl.ds(c * L, L)]
            better = (v > bv) | ((v == bv) & (ii < bi))
            for pj in _chosen_i:
                better = better & (ii != pj)
            bv = jnp.where(better, v, bv)
            bi = jnp.where(better, ii, bi)
            return (bv, bi)

        bv, bi = lax.fori_loop(0, n_slots, sel_body, (neg, imax_v))
        vj = jnp.max(bv)
        ij = jnp.min(jnp.where(bv == vj, bi, imax_v))
        chosen_v.append(vj)
        chosen_i.append(ij)

    v_vec = neg
    i_vec = zi
    for j in range(K):
        sel = lanes == j
        v_vec = jnp.where(sel, chosen_v[j], v_vec)
        i_vec = jnp.where(sel, chosen_i[j], i_vec)
    e_vec = jnp.exp((v_vec - mg) * INV_T)
    d_vec = s_total - (plsc.cumsum(e_vec) - e_vec)
    w_vec = e_vec / d_vec

    def zero_body(i, c):
        base = i * CHUNK
        for u in range(U):
            row[pl.ds(base + u * L, L)] = zf
        return c

    lax.fori_loop(0, N_STEPS, zero_body, 0)
    plsc.store_scatter(row, [i_vec], w_vec, mask=lanes < K)
    pltpu.sync_copy(row, out_hbm.at[wid])


def kernel(normu, cls):
    classes = _diff_topk_rows(cls)
    return (normu, classes)
